# baseline (speedup 1.0000x reference)
import functools

import jax
import jax.numpy as jnp
from jax import lax
from jax.experimental import pallas as pl
from jax.experimental.pallas import tpu as pltpu
from jax.experimental.pallas import tpu_sc as plsc

TOPK = 8
N_TOKENS = 32768
D_MODEL = 4096
N_EXPERTS = 64
BT = 1024

NCHUNK = 2
CT = N_TOKENS // NCHUNK
NW = 32
TPW = CT // NW


def _tc_body(x_ref, w_ref, probs_ref, logits_ref):
    x = x_ref[...]
    w = w_ref[...]
    logits = lax.dot_general(x, w, (((1,), (1,)), ((), ())),
                             preferred_element_type=jnp.float32)
    logits_ref[...] = logits
    lt = logits.T
    m = jnp.max(lt, axis=0, keepdims=True)
    et = jnp.exp(lt - m)
    s = jnp.sum(et, axis=0, keepdims=True)
    probs_ref[...] = (et / s).T


def _tc_chunk(x, W, c):
    nb = CT // BT
    return pl.pallas_call(
        _tc_body,
        grid=(nb,),
        in_specs=[
            pl.BlockSpec((BT, D_MODEL), lambda i, c=c: (i + c * nb, 0)),
            pl.BlockSpec((N_EXPERTS, D_MODEL), lambda i: (0, 0)),
        ],
        out_specs=(
            pl.BlockSpec((BT, N_EXPERTS), lambda i: (i, 0)),
            pl.BlockSpec((BT, N_EXPERTS), lambda i: (i, 0)),
        ),
        out_shape=(
            jax.ShapeDtypeStruct((CT, N_EXPERTS), jnp.float32),
            jax.ShapeDtypeStruct((CT, N_EXPERTS), jnp.float32),
        ),
    )(x, W)


def _merge(a, pa, b, pb):
    rb = lax.rev(b, (0,))
    rpb = lax.rev(pb, (0,))
    take = a >= rb
    m = jnp.where(take, a, rb)
    pm = jnp.where(take, pa, rpb)
    return plsc.sort_key_val(m, pm, descending=True)


@functools.partial(
    pl.kernel,
    mesh=plsc.VectorSubcoreMesh(core_axis_name="c", subcore_axis_name="s"),
    compiler_params=pltpu.CompilerParams(needs_layout_passes=False),
    out_type=(
        jax.ShapeDtypeStruct((CT * 16,), jnp.int32),
        jax.ShapeDtypeStruct((CT * 16,), jnp.float32),
    ),
    scratch_types=[
        pltpu.VMEM((TPW * N_EXPERTS,), jnp.float32),
        pltpu.VMEM((TPW * 16,), jnp.int32),
        pltpu.VMEM((TPW * 16,), jnp.float32),
        pltpu.SemaphoreType.DMA,
    ],
)
def _sc_topk(probs_hbm, idx_hbm, gate_hbm, pv, iv, gv, sem):
    wid = lax.axis_index("s") * 2 + lax.axis_index("c")
    base = wid * TPW
    pltpu.async_copy(probs_hbm.at[pl.ds(base * N_EXPERTS, TPW * N_EXPERTS)],
                     pv, sem).wait()

    lanes = lax.iota(jnp.int32, 16)
    first8 = lanes < TOPK
    pays = [lanes + 16 * j for j in range(4)]

    def one_token(t):
        sv = []
        for j in range(4):
            v = pv[pl.ds(t * N_EXPERTS + 16 * j, 16)]
            sv.append(plsc.sort_key_val(v, pays[j], descending=True))
        s01, p01 = _merge(sv[0][0], sv[0][1], sv[1][0], sv[1][1])
        s23, p23 = _merge(sv[2][0], sv[2][1], sv[3][0], sv[3][1])
        sf, pf = _merge(s01, p01, s23, p23)
        ssum = jnp.sum(jnp.where(first8, sf, 0.0))
        g16 = sf / lax.broadcast_in_dim(ssum, (16,), ())
        iv[pl.ds(t * 16, 16)] = pf
        gv[pl.ds(t * 16, 16)] = g16

    def body(i, carry):
        t0 = i * 4
        one_token(t0)
        one_token(t0 + 1)
        one_token(t0 + 2)
        one_token(t0 + 3)
        return carry

    lax.fori_loop(0, TPW // 4, body, 0)

    pltpu.sync_copy(iv, idx_hbm.at[pl.ds(base * 16, TPW * 16)])
    pltpu.sync_copy(gv, gate_hbm.at[pl.ds(base * 16, TPW * 16)])


@jax.jit
def kernel(x, W):
    ps, ls, iws, gws = [], [], [], []
    for c in range(NCHUNK):
        p, l = _tc_chunk(x, W, c)
        ps.append(p)
        ls.append(l)
    for c in range(NCHUNK):
        wi, wg = _sc_topk(ps[c].reshape(-1))
        iws.append(wi.reshape(CT, 16)[:, :TOPK])
        gws.append(wg.reshape(CT, 16)[:, :TOPK])
    probs = jnp.concatenate(ps, axis=0)
    logits = jnp.concatenate(ls, axis=0)
    topk_idx = jnp.concatenate(iws, axis=0)
    gate = jnp.concatenate(gws, axis=0)
    return (topk_idx, probs, gate, logits)


# speedup vs baseline: 1.3474x; 1.3474x over previous
import jax
import jax.numpy as jnp
from jax.experimental import pallas as pl

TOPK = 8
N_TOKENS = 32768
D_MODEL = 4096
N_EXPERTS = 64
BT = 1024


def _router_body(x_ref, w_ref, idx_ref, probs_ref, gate_ref, logits_ref):
    x = x_ref[...]
    w = w_ref[...]
    logits = jax.lax.dot_general(
        x, w, (((1,), (1,)), ((), ())),
        preferred_element_type=jnp.float32)
    logits_ref[...] = logits

    lt = logits.T
    m = jnp.max(lt, axis=0, keepdims=True)
    et = jnp.exp(lt - m)
    s = jnp.sum(et, axis=0, keepdims=True)
    probs_ref[...] = (et / s).T

    rows = jax.lax.broadcasted_iota(jnp.int32, et.shape, 0)
    work = et
    vals = []
    idxs = []
    for _ in range(TOPK):
        mx = jnp.max(work, axis=0, keepdims=True)
        ix = jnp.min(jnp.where(work == mx, rows, N_EXPERTS), axis=0,
                     keepdims=True)
        vals.append(mx)
        idxs.append(ix)
        work = jnp.where(rows == ix, -1.0, work)

    vals_t = jnp.concatenate(vals, axis=0)
    gate_t = vals_t / jnp.sum(vals_t, axis=0, keepdims=True)

    gate_ref[...] = gate_t.T
    idx_ref[...] = jnp.concatenate(idxs, axis=0).T


@jax.jit
def kernel(x, W):
    grid = (N_TOKENS // BT,)
    out_shapes = (
        jax.ShapeDtypeStruct((N_TOKENS, TOPK), jnp.int32),
        jax.ShapeDtypeStruct((N_TOKENS, N_EXPERTS), jnp.float32),
        jax.ShapeDtypeStruct((N_TOKENS, TOPK), jnp.float32),
        jax.ShapeDtypeStruct((N_TOKENS, N_EXPERTS), jnp.float32),
    )
    topk_idx, probs, gate, logits = pl.pallas_call(
        _router_body,
        grid=grid,
        in_specs=[
            pl.BlockSpec((BT, D_MODEL), lambda i: (i, 0)),
            pl.BlockSpec((N_EXPERTS, D_MODEL), lambda i: (0, 0)),
        ],
        out_specs=(
            pl.BlockSpec((BT, TOPK), lambda i: (i, 0)),
            pl.BlockSpec((BT, N_EXPERTS), lambda i: (i, 0)),
            pl.BlockSpec((BT, TOPK), lambda i: (i, 0)),
            pl.BlockSpec((BT, N_EXPERTS), lambda i: (i, 0)),
        ),
        out_shape=out_shapes,
    )(x, W)
    return (topk_idx, probs, gate, logits)
